# baseline (speedup 1.0000x reference)
import functools

import jax
import jax.numpy as jnp
from jax import lax
from jax.experimental import pallas as pl
from jax.experimental.pallas import tpu as pltpu
from jax.experimental.pallas import tpu_sc as plsc

EMBED = 64
NUM_CORES = 2
NUM_SUBCORES = 16
NW = NUM_CORES * NUM_SUBCORES
CHUNK = 128
NBUF = 5


@functools.cache
def _make_gather1(nchunk):
    assert nchunk % NBUF == 0 and nchunk > NBUF
    mesh = plsc.VectorSubcoreMesh(core_axis_name="c", subcore_axis_name="s")
    out_t = jax.ShapeDtypeStruct((NW, nchunk, CHUNK, EMBED), jnp.float32)

    @functools.partial(
        pl.kernel,
        mesh=mesh,
        out_type=out_t,
        compiler_params=pltpu.CompilerParams(use_tc_tiling_on_sc=False),
        scratch_types=[
            pltpu.VMEM((nchunk, CHUNK), jnp.int32),
            pltpu.VMEM((NBUF, CHUNK, EMBED), jnp.float32),
        ]
        + [pltpu.SemaphoreType.DMA] * (2 * NBUF),
    )
    def gather1(tok_idx, tab_in, out_ref, idx_v, rows_v, *sems):
        wid = lax.axis_index("s") * NUM_CORES + lax.axis_index("c")
        sem_g = sems[:NBUF]
        sem_w = sems[NBUF:]

        def fire(tab, b, c):
            pltpu.async_copy(tab.at[idx_v.at[c]], rows_v.at[b], sem_g[b])

        def drain(tab, b, c):
            pltpu.make_async_copy(
                tab.at[idx_v.at[c]], rows_v.at[b], sem_g[b]).wait()

        def put(out, b, c):
            pltpu.async_copy(rows_v.at[b], out.at[wid, c], sem_w[b])

        def put_wait(out, b, c):
            pltpu.make_async_copy(rows_v.at[b], out.at[wid, c], sem_w[b]).wait()

        def run_table(idx_hbm, tab, out):
            pltpu.sync_copy(idx_hbm.at[pl.ds(wid * nchunk, nchunk)], idx_v)
            for c in range(NBUF - 1):
                fire(tab, c, c)

            def step(i, carry):
                for b in range(NBUF):
                    c = i * NBUF + b
                    drain(tab, b, c)
                    put(out, b, c)
                    bf = (b + NBUF - 1) % NBUF

                    @pl.when(c + NBUF - 1 < nchunk)
                    def _():
                        @pl.when(c >= 1)
                        def _():
                            put_wait(out, bf, c - 1)

                        fire(tab, bf, c + NBUF - 1)
                return carry

            lax.fori_loop(0, nchunk // NBUF, step, 0)
            for b in range(NBUF):
                put_wait(out, b, nchunk - NBUF + b)

        run_table(tok_idx, tab_in, out_ref)

    return gather1


def kernel(num_tokens, unit_tokens, num_table, unit_table):
    B, S = num_tokens.shape
    rows = B * S
    assert rows % (NW * CHUNK) == 0
    nchunk = rows // (NW * CHUNK)
    half = S // 2
    nch = B * half // (NW * CHUNK)
    g = _make_gather1(nch)

    def run(tok, tab):
        parts = []
        for lo in (0, half):
            t = tok[:, lo:lo + half].reshape(NW * nch, CHUNK).astype(jnp.int32)
            parts.append(g(t, tab).reshape(B, half, EMBED))
        return jnp.concatenate(parts, axis=1)

    return (run(num_tokens, num_table), run(unit_tokens, unit_table))


# speedup vs baseline: 1.1921x; 1.1921x over previous
import functools

import jax
import jax.numpy as jnp
from jax import lax
from jax.experimental import pallas as pl
from jax.experimental.pallas import tpu as pltpu
from jax.experimental.pallas import tpu_sc as plsc

EMBED = 64
NUM_CORES = 2
NUM_SUBCORES = 16
NW = NUM_CORES * NUM_SUBCORES
CHUNK = 128
NBUF = 5


@functools.cache
def _make_gather1(nchunk):
    assert nchunk % NBUF == 0 and nchunk > NBUF
    mesh = plsc.VectorSubcoreMesh(core_axis_name="c", subcore_axis_name="s")
    out_t = jax.ShapeDtypeStruct((NW, nchunk, CHUNK, EMBED), jnp.float32)

    @functools.partial(
        pl.kernel,
        mesh=mesh,
        out_type=out_t,
        compiler_params=pltpu.CompilerParams(use_tc_tiling_on_sc=False),
        scratch_types=[
            pltpu.VMEM((nchunk, CHUNK), jnp.int32),
            pltpu.VMEM((NBUF, CHUNK, EMBED), jnp.float32),
        ]
        + [pltpu.SemaphoreType.DMA] * (2 * NBUF),
    )
    def gather1(tok_idx, tab_in, out_ref, idx_v, rows_v, *sems):
        wid = lax.axis_index("s") * NUM_CORES + lax.axis_index("c")
        sem_g = sems[:NBUF]
        sem_w = sems[NBUF:]

        def fire(tab, b, c):
            pltpu.async_copy(tab.at[idx_v.at[c]], rows_v.at[b], sem_g[b])

        def drain(tab, b, c):
            pltpu.make_async_copy(
                tab.at[idx_v.at[c]], rows_v.at[b], sem_g[b]).wait()

        def put(out, b, c):
            pltpu.async_copy(rows_v.at[b], out.at[wid, c], sem_w[b])

        def put_wait(out, b, c):
            pltpu.make_async_copy(rows_v.at[b], out.at[wid, c], sem_w[b]).wait()

        def run_table(idx_hbm, tab, out):
            pltpu.sync_copy(idx_hbm.at[pl.ds(wid * nchunk, nchunk)], idx_v)
            for c in range(NBUF - 1):
                fire(tab, c, c)

            def step(i, carry):
                for b in range(NBUF):
                    c = i * NBUF + b
                    drain(tab, b, c)
                    put(out, b, c)
                    bf = (b + NBUF - 1) % NBUF

                    @pl.when(c + NBUF - 1 < nchunk)
                    def _():
                        @pl.when(c >= 1)
                        def _():
                            put_wait(out, bf, c - 1)

                        fire(tab, bf, c + NBUF - 1)
                return carry

            lax.fori_loop(0, nchunk // NBUF, step, 0)
            for b in range(NBUF):
                put_wait(out, b, nchunk - NBUF + b)

        run_table(tok_idx, tab_in, out_ref)

    return gather1


def kernel(num_tokens, unit_tokens, num_table, unit_table):
    B, S = num_tokens.shape
    rows = B * S
    assert rows % (NW * CHUNK) == 0
    nchunk = rows // (NW * CHUNK)
    ni = num_tokens.reshape(NW * nchunk, CHUNK).astype(jnp.int32)
    ui = unit_tokens.reshape(NW * nchunk, CHUNK).astype(jnp.int32)
    g = _make_gather1(nchunk)
    out_num = g(ni, num_table)
    out_unit = g(ui, unit_table)
    return (out_num.reshape(B, S, EMBED), out_unit.reshape(B, S, EMBED))


# speedup vs baseline: 1.1939x; 1.0015x over previous
import functools

import jax
import jax.numpy as jnp
from jax import lax
from jax.experimental import pallas as pl
from jax.experimental.pallas import tpu as pltpu
from jax.experimental.pallas import tpu_sc as plsc

EMBED = 64
NUM_CORES = 2
NUM_SUBCORES = 16
NW = NUM_CORES * NUM_SUBCORES
CHUNK = 128
NBUF = 5


@functools.cache
def _make_gather1(nchunk):
    nsup = nchunk // 2
    assert nsup % NBUF == 0 and nsup > NBUF
    mesh = plsc.VectorSubcoreMesh(core_axis_name="c", subcore_axis_name="s")
    out_t = jax.ShapeDtypeStruct((NW, nsup, 2 * CHUNK, EMBED), jnp.float32)

    @functools.partial(
        pl.kernel,
        mesh=mesh,
        out_type=out_t,
        compiler_params=pltpu.CompilerParams(use_tc_tiling_on_sc=False),
        scratch_types=[
            pltpu.VMEM((nchunk, CHUNK), jnp.int32),
            pltpu.VMEM((NBUF, 2 * CHUNK, EMBED), jnp.float32),
        ]
        + [pltpu.SemaphoreType.DMA] * (2 * NBUF),
    )
    def gather1(tok_idx, tab_in, out_ref, idx_v, rows_v, *sems):
        wid = lax.axis_index("s") * NUM_CORES + lax.axis_index("c")
        sem_g = sems[:NBUF]
        sem_w = sems[NBUF:]

        def fire(tab, b, u):
            for h in range(2):
                pltpu.async_copy(
                    tab.at[idx_v.at[2 * u + h]],
                    rows_v.at[b, pl.ds(h * CHUNK, CHUNK)], sem_g[b])

        def drain(tab, b, u):
            for h in range(2):
                pltpu.make_async_copy(
                    tab.at[idx_v.at[2 * u + h]],
                    rows_v.at[b, pl.ds(h * CHUNK, CHUNK)], sem_g[b]).wait()

        def put(out, b, c):
            pltpu.async_copy(rows_v.at[b], out.at[wid, c], sem_w[b])

        def put_wait(out, b, c):
            pltpu.make_async_copy(rows_v.at[b], out.at[wid, c], sem_w[b]).wait()

        def run_table(idx_hbm, tab, out):
            pltpu.sync_copy(idx_hbm.at[pl.ds(wid * nchunk, nchunk)], idx_v)
            for c in range(NBUF - 1):
                fire(tab, c, c)

            def step(i, carry):
                for b in range(NBUF):
                    u = i * NBUF + b
                    drain(tab, b, u)
                    put(out, b, u)
                    bf = (b + NBUF - 1) % NBUF

                    @pl.when(u + NBUF - 1 < nsup)
                    def _():
                        @pl.when(u >= 1)
                        def _():
                            put_wait(out, bf, u - 1)

                        fire(tab, bf, u + NBUF - 1)
                return carry

            lax.fori_loop(0, nsup // NBUF, step, 0)
            for b in range(NBUF):
                put_wait(out, b, nsup - NBUF + b)

        run_table(tok_idx, tab_in, out_ref)

    return gather1


def kernel(num_tokens, unit_tokens, num_table, unit_table):
    B, S = num_tokens.shape
    rows = B * S
    assert rows % (NW * CHUNK) == 0
    nchunk = rows // (NW * CHUNK)
    ni = num_tokens.reshape(NW * nchunk, CHUNK).astype(jnp.int32)
    ui = unit_tokens.reshape(NW * nchunk, CHUNK).astype(jnp.int32)
    g = _make_gather1(nchunk)
    out_num = g(ni, num_table)
    out_unit = g(ui, unit_table)
    return (out_num.reshape(B, S, EMBED), out_unit.reshape(B, S, EMBED))
